# baseline (speedup 1.0000x reference)
import functools

import jax
import jax.numpy as jnp
from jax import lax
from jax.experimental import pallas as pl
from jax.experimental.pallas import tpu as pltpu
from jax.experimental.pallas import tpu_sc as plsc

NC = 2
NS = 16
LANES = 16
C = 80
RING = 4


def _sget(ref, k):
    return plsc.load_gather(ref, [jnp.full((LANES,), k, jnp.int32)])[0]


def _sc_body(nn, npairs, feat_hbm, dist_hbm, cos_hbm, node_hbm, i_hbm, j_hbm,
             zeros_hbm, out_hbm,
             nodeb, ib, jb, cosb, scidx, runs_v, d1_v, w_v, fi_v, fj_v,
             sc_v, rowbuf, bin_v, agg_sh,
             sem_fi, sem_fj, sem_idx, sem_row, sem_sc):
    d = fi_v.shape[1]
    cid = lax.axis_index("c")
    sid = lax.axis_index("s")
    half_n = nn // 2
    lane = jnp.arange(LANES, dtype=jnp.int32)

    rpt = (half_n // NS) // 8 * 8
    tail = half_n - rpt * NS
    row0 = sid * rpt
    pltpu.sync_copy(zeros_hbm.at[pl.ds(0, rpt)], agg_sh.at[pl.ds(row0, rpt)])

    @pl.when(sid == NS - 1)
    def _zero_tail():
        pltpu.sync_copy(zeros_hbm.at[pl.ds(0, tail)],
                        agg_sh.at[pl.ds(rpt * NS, tail)])

    def bs_body(_, lohi):
        lo, hi = lohi
        cont = lo < hi
        mid = (lo + hi) >> 1
        off = jnp.minimum((mid >> 3) << 3, npairs - LANES)
        pltpu.sync_copy(node_hbm.at[pl.ds(pl.multiple_of(off, 8), LANES)],
                        bin_v)
        val = _sget(bin_v, mid - off)
        lo2 = jnp.where(val < half_n, mid + 1, lo)
        hi2 = jnp.where(val < half_n, hi, mid)
        return (jnp.where(cont, lo2, lo), jnp.where(cont, hi2, hi))

    bnd, _ = lax.fori_loop(0, 18, bs_body,
                           (jnp.int32(0), jnp.int32(npairs)))

    lo_sc = jnp.where(cid == 0, 0, (bnd >> 3) << 3)
    hi_sc = jnp.where(cid == 0,
                      jnp.minimum(((bnd + 7) >> 3) << 3, npairs), npairs)
    span = hi_sc - lo_sc
    sz = (span + NS - 1) // NS

    def bound(s):
        return (jnp.minimum(lo_sc + s * sz, hi_sc) >> 3) << 3

    start = bound(sid)
    end = jnp.where(sid == NS - 1, hi_sc, bound(sid + 1))
    nchunks = (end - start + C - 1) // C
    node_base = cid * half_n

    def stage_idx(c):
        par = c & 1
        b = pl.multiple_of(jnp.minimum(start + c * C, npairs - C), 8)
        pltpu.async_copy(node_hbm.at[pl.ds(b, C)], nodeb.at[par], sem_idx)
        pltpu.async_copy(i_hbm.at[pl.ds(b, C)], ib.at[par], sem_idx)
        pltpu.async_copy(j_hbm.at[pl.ds(b, C)], jb.at[par], sem_idx)
        pltpu.async_copy(cos_hbm.at[pl.ds(b, C)], cosb.at[par], sem_idx)

    def wait_idx(c):
        par = c & 1
        b = pl.multiple_of(jnp.minimum(start + c * C, npairs - C), 8)
        pltpu.make_async_copy(node_hbm.at[pl.ds(b, C)], nodeb.at[par],
                              sem_idx).wait()
        pltpu.make_async_copy(i_hbm.at[pl.ds(b, C)], ib.at[par],
                              sem_idx).wait()
        pltpu.make_async_copy(j_hbm.at[pl.ds(b, C)], jb.at[par],
                              sem_idx).wait()
        pltpu.make_async_copy(cos_hbm.at[pl.ds(b, C)], cosb.at[par],
                              sem_idx).wait()

    def build_runs(c):
        par = c & 1
        nrun = jnp.int32(0)
        for g in range(C // LANES):
            s = pl.ds(g * LANES, LANES)
            nodes_g = nodeb[par, s]
            if g == 0:
                prev = plsc.load_gather(
                    nodeb.at[par], [jnp.maximum(lane - 1, 0)])
                m = (nodes_g != prev) | (lane == 0)
            else:
                prev = nodeb[par, pl.ds(g * LANES - 1, LANES)]
                m = nodes_g != prev
            plsc.store_compressed(runs_v.at[par, pl.ds(nrun, LANES)],
                                  nodes_g, mask=m)
            nrun = nrun + plsc.all_reduce_population_count(m)[0]
        return nrun

    def row_dma(c, k, slot):
        row = _sget(runs_v.at[c & 1], k)
        return pltpu.async_copy(dist_hbm.at[row], rowbuf.at[slot], sem_row)

    def fire_rows(c, nrun):
        for kk in range(RING):
            @pl.when(kk < nrun)
            def _(kk=kk):
                row_dma(c, jnp.int32(kk), jnp.int32(kk))

    @pl.when(nchunks > 0)
    def _stage0():
        stage_idx(jnp.int32(0))

    plsc.subcore_barrier()

    def _pro2():
        wait_idx(jnp.int32(0))
        nr0 = build_runs(jnp.int32(0))
        fire_rows(jnp.int32(0), nr0)

        @pl.when(nchunks > 1)
        def _():
            stage_idx(jnp.int32(1))

        return nr0

    nrun0 = lax.cond(nchunks > 0, _pro2, lambda: jnp.int32(0))

    def chunk_body(c, nrun_c):
        par = c & 1
        nb = nodeb.at[par]
        ibr = ib.at[par]
        jbr = jb.at[par]
        cp_fi = pltpu.async_copy(feat_hbm.at[ibr], fi_v, sem_fi)
        cp_fj = pltpu.async_copy(feat_hbm.at[jbr], fj_v, sem_fj)

        def _prep_next():
            with jax.named_scope("phase_idxwait"):
                wait_idx(c + 1)
            return build_runs(c + 1)

        nrun_next = lax.cond(c + 1 < nchunks, _prep_next,
                             lambda: jnp.int32(0))

        def run_body(k, carry2):
            slot = k & (RING - 1)
            row = _sget(runs_v.at[par], k)
            pltpu.make_async_copy(dist_hbm.at[row], rowbuf.at[slot],
                                  sem_row).wait()
            rowsel = jnp.full((LANES,), slot, jnp.int32)
            for g in range(C // LANES):
                s = pl.ds(g * LANES, LANES)
                m = nb[s] == row
                d1g = plsc.load_gather(rowbuf, [rowsel, ibr[s]])
                d2g = plsc.load_gather(rowbuf, [rowsel, jbr[s]])
                d1_v[s] = jnp.where(m, d1g, d1_v[s])
                w_v[s] = jnp.where(m, d2g, w_v[s])

            @pl.when(k + RING < nrun_c)
            def _():
                row_dma(c, k + RING, slot)

            return carry2

        with jax.named_scope("phase_rows"):
            lax.fori_loop(0, nrun_c, run_body, 0)

        @pl.when(c + 1 < nchunks)
        def _():
            fire_rows(c + 1, nrun_next)

        b = jnp.minimum(start + c * C, npairs - C)
        b_raw = start + c * C
        cbr = cosb.at[par]
        sxr = scidx.at[par]
        for g in range(C // LANES):
            s = pl.ds(g * LANES, LANES)
            pos = b + (g * LANES) + lane
            rel = nb[s] - node_base
            valid = ((pos >= b_raw) & (pos < end)
                     & (rel >= 0) & (rel < half_n))
            w_v[s] = jnp.where(valid, cbr[s] * d1_v[s] * w_v[s], 0.0)
            sxr[s] = jnp.clip(rel, 0, half_n - 1)

        with jax.named_scope("phase_featwait"):
            cp_fi.wait()
            cp_fj.wait()

        @pl.when(c + 2 < nchunks)
        def _():
            stage_idx(c + 2)

        @pl.when(c > 0)
        def _():
            parp = 1 - par
            pltpu.make_async_copy(
                sc_v.at[parp], agg_sh.at[scidx.at[parp]], sem_sc).wait()

        with jax.named_scope("phase_scale"):
            @plsc.parallel_loop(0, C, unroll=8)
            def scale_row(p):
                fir = fi_v.at[p]
                fjr = fj_v.at[p]
                scr = sc_v.at[par, p]
                wv = plsc.load_gather(w_v,
                                      [jnp.full((LANES,), p, jnp.int32)])
                for k in range(d // LANES):
                    s = pl.ds(k * LANES, LANES)
                    scr[s] = (fir[s] + fjr[s]) * wv
        pltpu.async_copy(sc_v.at[par], agg_sh.at[scidx.at[par]], sem_sc,
                         add=True)
        return nrun_next

    lax.fori_loop(0, nchunks, chunk_body, nrun0)

    @pl.when(nchunks > 0)
    def _drain():
        parl = (nchunks - 1) & 1
        pltpu.make_async_copy(sc_v.at[parl], agg_sh.at[scidx.at[parl]],
                              sem_sc).wait()

    plsc.subcore_barrier()
    out_base = pl.multiple_of(node_base + row0, 8)
    pltpu.sync_copy(agg_sh.at[pl.ds(row0, rpt)],
                    out_hbm.at[pl.ds(out_base, rpt)])

    @pl.when(sid == NS - 1)
    def _dump_tail():
        pltpu.sync_copy(
            agg_sh.at[pl.ds(rpt * NS, tail)],
            out_hbm.at[pl.ds(pl.multiple_of(node_base + rpt * NS, 8), tail)])


def _proj_body(agg_ref, w_ref, b_ref, o_ref):
    o_ref[...] = (jnp.dot(agg_ref[...], w_ref[...],
                          preferred_element_type=jnp.float32) + b_ref[...])


def kernel(features, dist, cos_vals, W, b, node_idx, pair_i, pair_j):
    n, d = features.shape
    p = cos_vals.shape[0]
    zeros = jnp.zeros(((n // 2 // NS) // 8 * 8, d), jnp.float32)

    sc_fn = pl.kernel(
        functools.partial(_sc_body, n, p),
        out_type=jax.ShapeDtypeStruct((n, d), jnp.float32),
        mesh=plsc.VectorSubcoreMesh(
            core_axis_name="c", subcore_axis_name="s",
            num_cores=NC, num_subcores=NS),
        compiler_params=pltpu.CompilerParams(needs_layout_passes=False),
        scratch_types=[
            pltpu.VMEM((2, C), jnp.int32),
            pltpu.VMEM((2, C), jnp.int32),
            pltpu.VMEM((2, C), jnp.int32),
            pltpu.VMEM((2, C), jnp.float32),
            pltpu.VMEM((2, C), jnp.int32),
            pltpu.VMEM((2, C + LANES), jnp.int32),
            pltpu.VMEM((C,), jnp.float32),
            pltpu.VMEM((C,), jnp.float32),
            pltpu.VMEM((C, d), jnp.float32),
            pltpu.VMEM((C, d), jnp.float32),
            pltpu.VMEM((2, C, d), jnp.float32),
            pltpu.VMEM((RING, n), jnp.float32),
            pltpu.VMEM((LANES,), jnp.int32),
            pltpu.VMEM_SHARED((n // 2, d), jnp.float32),
            pltpu.SemaphoreType.DMA,
            pltpu.SemaphoreType.DMA,
            pltpu.SemaphoreType.DMA,
            pltpu.SemaphoreType.DMA,
            pltpu.SemaphoreType.DMA,
        ],
    )
    agg = sc_fn(features, dist, cos_vals, node_idx, pair_i, pair_j, zeros)

    blk = 1000
    out = pl.pallas_call(
        _proj_body,
        grid=(n // blk,),
        in_specs=[
            pl.BlockSpec((blk, d), lambda i: (i, 0)),
            pl.BlockSpec((d, d), lambda i: (0, 0)),
            pl.BlockSpec((1, d), lambda i: (0, 0)),
        ],
        out_specs=pl.BlockSpec((blk, d), lambda i: (i, 0)),
        out_shape=jax.ShapeDtypeStruct((n, d), jnp.float32),
    )(agg, W, b.reshape(1, d))
    return out


# speedup vs baseline: 1.0082x; 1.0082x over previous
import functools

import jax
import jax.numpy as jnp
from jax import lax
from jax.experimental import pallas as pl
from jax.experimental.pallas import tpu as pltpu
from jax.experimental.pallas import tpu_sc as plsc

NC = 2
NS = 16
LANES = 16
C = 80
RING = 4


def _sget(ref, k):
    return plsc.load_gather(ref, [jnp.full((LANES,), k, jnp.int32)])[0]


def _sc_body(nn, npairs, feat_hbm, dist_hbm, cos_hbm, node_hbm, i_hbm, j_hbm,
             zeros_hbm, out_hbm,
             nodeb, ib, jb, cosb, scidx, runs_v, d1_v, w_v, fi_v, fj_v,
             sc_v, rowbuf, bin_v, agg_sh,
             sem_fi, sem_fj, sem_idx, sem_row, sem_sc):
    d = fi_v.shape[1]
    cid = lax.axis_index("c")
    sid = lax.axis_index("s")
    half_n = nn // 2
    lane = jnp.arange(LANES, dtype=jnp.int32)

    rpt = (half_n // NS) // 8 * 8
    tail = half_n - rpt * NS
    row0 = sid * rpt
    pltpu.sync_copy(zeros_hbm.at[pl.ds(0, rpt)], agg_sh.at[pl.ds(row0, rpt)])

    @pl.when(sid == NS - 1)
    def _zero_tail():
        pltpu.sync_copy(zeros_hbm.at[pl.ds(0, tail)],
                        agg_sh.at[pl.ds(rpt * NS, tail)])

    def bs_body(_, lohi):
        lo, hi = lohi
        cont = lo < hi
        mid = (lo + hi) >> 1
        off = jnp.minimum((mid >> 3) << 3, npairs - LANES)
        pltpu.sync_copy(node_hbm.at[pl.ds(pl.multiple_of(off, 8), LANES)],
                        bin_v)
        val = _sget(bin_v, mid - off)
        lo2 = jnp.where(val < half_n, mid + 1, lo)
        hi2 = jnp.where(val < half_n, hi, mid)
        return (jnp.where(cont, lo2, lo), jnp.where(cont, hi2, hi))

    bnd, _ = lax.fori_loop(0, 18, bs_body,
                           (jnp.int32(0), jnp.int32(npairs)))

    lo_sc = jnp.where(cid == 0, 0, (bnd >> 3) << 3)
    hi_sc = jnp.where(cid == 0,
                      jnp.minimum(((bnd + 7) >> 3) << 3, npairs), npairs)
    span = hi_sc - lo_sc
    sz = (span + NS - 1) // NS

    def bound(s):
        return (jnp.minimum(lo_sc + s * sz, hi_sc) >> 3) << 3

    start = bound(sid)
    end = jnp.where(sid == NS - 1, hi_sc, bound(sid + 1))
    nchunks = (end - start + C - 1) // C
    node_base = cid * half_n

    def stage_idx(c):
        par = c & 1
        b = pl.multiple_of(jnp.minimum(start + c * C, npairs - C), 8)
        pltpu.async_copy(node_hbm.at[pl.ds(b, C)], nodeb.at[par], sem_idx)
        pltpu.async_copy(i_hbm.at[pl.ds(b, C)], ib.at[par], sem_idx)
        pltpu.async_copy(j_hbm.at[pl.ds(b, C)], jb.at[par], sem_idx)
        pltpu.async_copy(cos_hbm.at[pl.ds(b, C)], cosb.at[par], sem_idx)

    def wait_idx(c):
        par = c & 1
        b = pl.multiple_of(jnp.minimum(start + c * C, npairs - C), 8)
        pltpu.make_async_copy(node_hbm.at[pl.ds(b, C)], nodeb.at[par],
                              sem_idx).wait()
        pltpu.make_async_copy(i_hbm.at[pl.ds(b, C)], ib.at[par],
                              sem_idx).wait()
        pltpu.make_async_copy(j_hbm.at[pl.ds(b, C)], jb.at[par],
                              sem_idx).wait()
        pltpu.make_async_copy(cos_hbm.at[pl.ds(b, C)], cosb.at[par],
                              sem_idx).wait()

    def build_runs(c):
        par = c & 1
        nrun = jnp.int32(0)
        for g in range(C // LANES):
            s = pl.ds(g * LANES, LANES)
            nodes_g = nodeb[par, s]
            if g == 0:
                prev = plsc.load_gather(
                    nodeb.at[par], [jnp.maximum(lane - 1, 0)])
                m = (nodes_g != prev) | (lane == 0)
            else:
                prev = nodeb[par, pl.ds(g * LANES - 1, LANES)]
                m = nodes_g != prev
            plsc.store_compressed(runs_v.at[par, pl.ds(nrun, LANES)],
                                  nodes_g, mask=m)
            nrun = nrun + plsc.all_reduce_population_count(m)[0]
        return nrun

    def row_dma(c, k, slot):
        row = _sget(runs_v.at[c & 1], k)
        return pltpu.async_copy(dist_hbm.at[row], rowbuf.at[slot], sem_row)

    def fire_rows(c, nrun):
        for kk in range(RING):
            @pl.when(kk < nrun)
            def _(kk=kk):
                row_dma(c, jnp.int32(kk), jnp.int32(kk))

    @pl.when(nchunks > 0)
    def _stage0():
        stage_idx(jnp.int32(0))

    plsc.subcore_barrier()

    def _pro2():
        wait_idx(jnp.int32(0))
        nr0 = build_runs(jnp.int32(0))
        fire_rows(jnp.int32(0), nr0)

        @pl.when(nchunks > 1)
        def _():
            stage_idx(jnp.int32(1))

        return nr0

    nrun0 = lax.cond(nchunks > 0, _pro2, lambda: jnp.int32(0))

    def chunk_body(c, nrun_c):
        par = c & 1
        nb = nodeb.at[par]
        ibr = ib.at[par]
        jbr = jb.at[par]
        cp_fi = pltpu.async_copy(feat_hbm.at[ibr], fi_v, sem_fi)
        cp_fj = pltpu.async_copy(feat_hbm.at[jbr], fj_v, sem_fj)

        def _prep_next():
            with jax.named_scope("phase_idxwait"):
                wait_idx(c + 1)
            return build_runs(c + 1)

        nrun_next = lax.cond(c + 1 < nchunks, _prep_next,
                             lambda: jnp.int32(0))

        def run_body(k, carry2):
            slot = k & (RING - 1)
            row = _sget(runs_v.at[par], k)
            pltpu.make_async_copy(dist_hbm.at[row], rowbuf.at[slot],
                                  sem_row).wait()
            rowsel = jnp.full((LANES,), slot, jnp.int32)
            for g in range(C // LANES):
                s = pl.ds(g * LANES, LANES)
                m = nb[s] == row
                d1g = plsc.load_gather(rowbuf, [rowsel, ibr[s]])
                d2g = plsc.load_gather(rowbuf, [rowsel, jbr[s]])
                d1_v[s] = jnp.where(m, d1g, d1_v[s])
                w_v[s] = jnp.where(m, d2g, w_v[s])

            @pl.when(k + RING < nrun_c)
            def _():
                row_dma(c, k + RING, slot)

            return carry2

        with jax.named_scope("phase_rows"):
            lax.fori_loop(0, nrun_c, run_body, 0)

        @pl.when(c + 1 < nchunks)
        def _():
            fire_rows(c + 1, nrun_next)

        b = jnp.minimum(start + c * C, npairs - C)
        b_raw = start + c * C
        cbr = cosb.at[par]
        sxr = scidx.at[par]
        for g in range(C // LANES):
            s = pl.ds(g * LANES, LANES)
            pos = b + (g * LANES) + lane
            rel = nb[s] - node_base
            valid = ((pos >= b_raw) & (pos < end)
                     & (rel >= 0) & (rel < half_n))
            w_v[s] = jnp.where(valid, cbr[s] * d1_v[s] * w_v[s], 0.0)
            sxr[s] = jnp.clip(rel, 0, half_n - 1)

        with jax.named_scope("phase_featwait"):
            cp_fi.wait()
            cp_fj.wait()

        @pl.when(c + 2 < nchunks)
        def _():
            stage_idx(c + 2)

        @pl.when(c > 0)
        def _():
            parp = 1 - par
            pltpu.make_async_copy(
                sc_v.at[parp], agg_sh.at[scidx.at[parp]], sem_sc).wait()

        with jax.named_scope("phase_scale"):
            @plsc.parallel_loop(0, C, unroll=4)
            def scale_row(p):
                fir = fi_v.at[p]
                fjr = fj_v.at[p]
                scr = sc_v.at[par, p]
                wv = plsc.load_gather(w_v,
                                      [jnp.full((LANES,), p, jnp.int32)])
                for k in range(d // LANES):
                    s = pl.ds(k * LANES, LANES)
                    scr[s] = (fir[s] + fjr[s]) * wv
        pltpu.async_copy(sc_v.at[par], agg_sh.at[scidx.at[par]], sem_sc,
                         add=True)
        return nrun_next

    lax.fori_loop(0, nchunks, chunk_body, nrun0)

    @pl.when(nchunks > 0)
    def _drain():
        parl = (nchunks - 1) & 1
        pltpu.make_async_copy(sc_v.at[parl], agg_sh.at[scidx.at[parl]],
                              sem_sc).wait()

    plsc.subcore_barrier()
    out_base = pl.multiple_of(node_base + row0, 8)
    pltpu.sync_copy(agg_sh.at[pl.ds(row0, rpt)],
                    out_hbm.at[pl.ds(out_base, rpt)])

    @pl.when(sid == NS - 1)
    def _dump_tail():
        pltpu.sync_copy(
            agg_sh.at[pl.ds(rpt * NS, tail)],
            out_hbm.at[pl.ds(pl.multiple_of(node_base + rpt * NS, 8), tail)])


def _proj_body(agg_ref, w_ref, b_ref, o_ref):
    o_ref[...] = (jnp.dot(agg_ref[...], w_ref[...],
                          preferred_element_type=jnp.float32) + b_ref[...])


def kernel(features, dist, cos_vals, W, b, node_idx, pair_i, pair_j):
    n, d = features.shape
    p = cos_vals.shape[0]
    zeros = jnp.zeros(((n // 2 // NS) // 8 * 8, d), jnp.float32)

    sc_fn = pl.kernel(
        functools.partial(_sc_body, n, p),
        out_type=jax.ShapeDtypeStruct((n, d), jnp.float32),
        mesh=plsc.VectorSubcoreMesh(
            core_axis_name="c", subcore_axis_name="s",
            num_cores=NC, num_subcores=NS),
        compiler_params=pltpu.CompilerParams(needs_layout_passes=False),
        scratch_types=[
            pltpu.VMEM((2, C), jnp.int32),
            pltpu.VMEM((2, C), jnp.int32),
            pltpu.VMEM((2, C), jnp.int32),
            pltpu.VMEM((2, C), jnp.float32),
            pltpu.VMEM((2, C), jnp.int32),
            pltpu.VMEM((2, C + LANES), jnp.int32),
            pltpu.VMEM((C,), jnp.float32),
            pltpu.VMEM((C,), jnp.float32),
            pltpu.VMEM((C, d), jnp.float32),
            pltpu.VMEM((C, d), jnp.float32),
            pltpu.VMEM((2, C, d), jnp.float32),
            pltpu.VMEM((RING, n), jnp.float32),
            pltpu.VMEM((LANES,), jnp.int32),
            pltpu.VMEM_SHARED((n // 2, d), jnp.float32),
            pltpu.SemaphoreType.DMA,
            pltpu.SemaphoreType.DMA,
            pltpu.SemaphoreType.DMA,
            pltpu.SemaphoreType.DMA,
            pltpu.SemaphoreType.DMA,
        ],
    )
    agg = sc_fn(features, dist, cos_vals, node_idx, pair_i, pair_j, zeros)

    blk = 1000
    out = pl.pallas_call(
        _proj_body,
        grid=(n // blk,),
        in_specs=[
            pl.BlockSpec((blk, d), lambda i: (i, 0)),
            pl.BlockSpec((d, d), lambda i: (0, 0)),
            pl.BlockSpec((1, d), lambda i: (0, 0)),
        ],
        out_specs=pl.BlockSpec((blk, d), lambda i: (i, 0)),
        out_shape=jax.ShapeDtypeStruct((n, d), jnp.float32),
    )(agg, W, b.reshape(1, d))
    return out


# speedup vs baseline: 1.0762x; 1.0675x over previous
import functools

import jax
import jax.numpy as jnp
from jax import lax
from jax.experimental import pallas as pl
from jax.experimental.pallas import tpu as pltpu
from jax.experimental.pallas import tpu_sc as plsc

NC = 2
NS = 16
LANES = 16
C = 80
RING = 4


def _sget(ref, k):
    return plsc.load_gather(ref, [jnp.full((LANES,), k, jnp.int32)])[0]


def _sc_body(nn, npairs, feat_hbm, dist_hbm, cos_hbm, node_hbm, i_hbm, j_hbm,
             zeros_hbm, out_hbm,
             nodeb, ib, jb, cosb, scidx, runs_v, d1_v, w_v, fi_v, fj_v,
             sc_v, rowbuf, bin_v, agg_sh,
             sem_fi, sem_fj, sem_idx, sem_row, sem_sc):
    d = fi_v.shape[1]
    cid = lax.axis_index("c")
    sid = lax.axis_index("s")
    half_n = nn // 2
    lane = jnp.arange(LANES, dtype=jnp.int32)

    rpt = (half_n // NS) // 8 * 8
    tail = half_n - rpt * NS
    row0 = sid * rpt
    pltpu.sync_copy(zeros_hbm.at[pl.ds(0, rpt)], agg_sh.at[pl.ds(row0, rpt)])

    @pl.when(sid == NS - 1)
    def _zero_tail():
        pltpu.sync_copy(zeros_hbm.at[pl.ds(0, tail)],
                        agg_sh.at[pl.ds(rpt * NS, tail)])

    def bs_body(_, lohi):
        lo, hi = lohi
        cont = lo < hi
        mid = (lo + hi) >> 1
        off = jnp.minimum((mid >> 3) << 3, npairs - LANES)
        pltpu.sync_copy(node_hbm.at[pl.ds(pl.multiple_of(off, 8), LANES)],
                        bin_v)
        val = _sget(bin_v, mid - off)
        lo2 = jnp.where(val < half_n, mid + 1, lo)
        hi2 = jnp.where(val < half_n, hi, mid)
        return (jnp.where(cont, lo2, lo), jnp.where(cont, hi2, hi))

    bnd, _ = lax.fori_loop(0, 18, bs_body,
                           (jnp.int32(0), jnp.int32(npairs)))

    lo_sc = jnp.where(cid == 0, 0, (bnd >> 3) << 3)
    hi_sc = jnp.where(cid == 0,
                      jnp.minimum(((bnd + 7) >> 3) << 3, npairs), npairs)
    span = hi_sc - lo_sc
    sz = (span + NS - 1) // NS

    def bound(s):
        return (jnp.minimum(lo_sc + s * sz, hi_sc) >> 3) << 3

    start = bound(sid)
    end = jnp.where(sid == NS - 1, hi_sc, bound(sid + 1))
    nchunks = (end - start + C - 1) // C
    node_base = cid * half_n

    def stage_idx(c):
        par = c & 1
        b = pl.multiple_of(jnp.minimum(start + c * C, npairs - C), 8)
        pltpu.async_copy(node_hbm.at[pl.ds(b, C)], nodeb.at[par], sem_idx)
        pltpu.async_copy(i_hbm.at[pl.ds(b, C)], ib.at[par], sem_idx)
        pltpu.async_copy(j_hbm.at[pl.ds(b, C)], jb.at[par], sem_idx)
        pltpu.async_copy(cos_hbm.at[pl.ds(b, C)], cosb.at[par], sem_idx)

    def wait_idx(c):
        par = c & 1
        b = pl.multiple_of(jnp.minimum(start + c * C, npairs - C), 8)
        pltpu.make_async_copy(node_hbm.at[pl.ds(b, C)], nodeb.at[par],
                              sem_idx).wait()
        pltpu.make_async_copy(i_hbm.at[pl.ds(b, C)], ib.at[par],
                              sem_idx).wait()
        pltpu.make_async_copy(j_hbm.at[pl.ds(b, C)], jb.at[par],
                              sem_idx).wait()
        pltpu.make_async_copy(cos_hbm.at[pl.ds(b, C)], cosb.at[par],
                              sem_idx).wait()

    def build_runs(c, reuse):
        par = c & 1
        nrun = jnp.int32(0)
        for g in range(C // LANES):
            s = pl.ds(g * LANES, LANES)
            nodes_g = nodeb[par, s]
            if g == 0:
                prev = plsc.load_gather(
                    nodeb.at[par], [jnp.maximum(lane - 1, 0)])
                m = (nodes_g != prev) | ((lane == 0) & (reuse == 0))
            else:
                prev = nodeb[par, pl.ds(g * LANES - 1, LANES)]
                m = nodes_g != prev
            plsc.store_compressed(runs_v.at[par, pl.ds(nrun, LANES)],
                                  nodes_g, mask=m)
            nrun = nrun + plsc.all_reduce_population_count(m)[0]
        return nrun

    def row_dma(c, klist, slot):
        row = _sget(runs_v.at[c & 1], klist)
        return pltpu.async_copy(dist_hbm.at[row], rowbuf.at[slot], sem_row)

    def fire_rows(c, nrun, reuse, g4):
        for kk in range(RING):
            @pl.when((kk >= reuse) & (kk < nrun + reuse))
            def _(kk=kk):
                row_dma(c, jnp.int32(kk) - reuse, (g4 + kk) & (RING - 1))

    @pl.when(nchunks > 0)
    def _stage0():
        stage_idx(jnp.int32(0))

    plsc.subcore_barrier()

    def _pro2():
        wait_idx(jnp.int32(0))
        nr0 = build_runs(jnp.int32(0), jnp.int32(0))
        fire_rows(jnp.int32(0), nr0, jnp.int32(0), jnp.int32(0))

        @pl.when(nchunks > 1)
        def _():
            stage_idx(jnp.int32(1))

        return nr0

    nrun0 = lax.cond(nchunks > 0, _pro2, lambda: jnp.int32(0))

    def chunk_body(c, carry):
        nrun_c, reuse_c, g4_c, prevl_c = carry
        par = c & 1
        nb = nodeb.at[par]
        ibr = ib.at[par]
        jbr = jb.at[par]
        cp_fi = pltpu.async_copy(feat_hbm.at[ibr], fi_v, sem_fi)
        cp_fj = pltpu.async_copy(feat_hbm.at[jbr], fj_v, sem_fj)

        lastc = nb[pl.ds(C - LANES, LANES)][LANES - 1]

        def _prep_next():
            with jax.named_scope("phase_idxwait"):
                wait_idx(c + 1)
            parn = 1 - par
            firstn = nodeb[parn, pl.ds(0, LANES)][0]
            bc = jnp.minimum(start + c * C, npairs - C)
            bn = jnp.minimum(start + (c + 1) * C, npairs - C)
            reuse_n = jnp.where((bn == bc + C) & (firstn == lastc),
                                jnp.int32(1), jnp.int32(0))
            return build_runs(c + 1, reuse_n), reuse_n

        nrun_next, reuse_next = lax.cond(
            c + 1 < nchunks, _prep_next,
            lambda: (jnp.int32(0), jnp.int32(0)))

        nloc = nrun_c + reuse_c

        def run_body(k, carry2):
            slot = (g4_c + k) & (RING - 1)
            row = jnp.where(reuse_c & (k == 0), prevl_c,
                            _sget(runs_v.at[par], k - reuse_c))

            @pl.when(k >= reuse_c)
            def _():
                pltpu.make_async_copy(dist_hbm.at[row], rowbuf.at[slot],
                                      sem_row).wait()

            rowsel = jnp.full((LANES,), slot, jnp.int32)
            for g in range(C // LANES):
                s = pl.ds(g * LANES, LANES)
                m = nb[s] == row
                d1g = plsc.load_gather(rowbuf, [rowsel, ibr[s]])
                d2g = plsc.load_gather(rowbuf, [rowsel, jbr[s]])
                d1_v[s] = jnp.where(m, d1g, d1_v[s])
                w_v[s] = jnp.where(m, d2g, w_v[s])

            @pl.when(k + RING < nloc)
            def _():
                row_dma(c, k + RING - reuse_c, slot)

            return carry2

        with jax.named_scope("phase_rows"):
            lax.fori_loop(0, nloc, run_body, 0)

        g4_next = (g4_c + nloc - reuse_next) & (RING - 1)

        @pl.when(c + 1 < nchunks)
        def _():
            fire_rows(c + 1, nrun_next, reuse_next, g4_next)

        b = jnp.minimum(start + c * C, npairs - C)
        b_raw = start + c * C
        cbr = cosb.at[par]
        sxr = scidx.at[par]
        for g in range(C // LANES):
            s = pl.ds(g * LANES, LANES)
            pos = b + (g * LANES) + lane
            rel = nb[s] - node_base
            valid = ((pos >= b_raw) & (pos < end)
                     & (rel >= 0) & (rel < half_n))
            w_v[s] = jnp.where(valid, cbr[s] * d1_v[s] * w_v[s], 0.0)
            sxr[s] = jnp.clip(rel, 0, half_n - 1)

        with jax.named_scope("phase_featwait"):
            cp_fi.wait()
            cp_fj.wait()

        @pl.when(c + 2 < nchunks)
        def _():
            stage_idx(c + 2)

        @pl.when(c > 0)
        def _():
            parp = 1 - par
            pltpu.make_async_copy(
                sc_v.at[parp], agg_sh.at[scidx.at[parp]], sem_sc).wait()

        with jax.named_scope("phase_scale"):
            @plsc.parallel_loop(0, C, unroll=4)
            def scale_row(p):
                fir = fi_v.at[p]
                fjr = fj_v.at[p]
                scr = sc_v.at[par, p]
                wv = plsc.load_gather(w_v,
                                      [jnp.full((LANES,), p, jnp.int32)])
                for k in range(d // LANES):
                    s = pl.ds(k * LANES, LANES)
                    scr[s] = (fir[s] + fjr[s]) * wv
        pltpu.async_copy(sc_v.at[par], agg_sh.at[scidx.at[par]], sem_sc,
                         add=True)
        return (nrun_next, reuse_next, g4_next, lastc)

    lax.fori_loop(0, nchunks, chunk_body,
                  (nrun0, jnp.int32(0), jnp.int32(0), jnp.int32(-1)))

    @pl.when(nchunks > 0)
    def _drain():
        parl = (nchunks - 1) & 1
        pltpu.make_async_copy(sc_v.at[parl], agg_sh.at[scidx.at[parl]],
                              sem_sc).wait()

    plsc.subcore_barrier()
    out_base = pl.multiple_of(node_base + row0, 8)
    pltpu.sync_copy(agg_sh.at[pl.ds(row0, rpt)],
                    out_hbm.at[pl.ds(out_base, rpt)])

    @pl.when(sid == NS - 1)
    def _dump_tail():
        pltpu.sync_copy(
            agg_sh.at[pl.ds(rpt * NS, tail)],
            out_hbm.at[pl.ds(pl.multiple_of(node_base + rpt * NS, 8), tail)])


def _proj_body(agg_ref, w_ref, b_ref, o_ref):
    o_ref[...] = (jnp.dot(agg_ref[...], w_ref[...],
                          preferred_element_type=jnp.float32) + b_ref[...])


def kernel(features, dist, cos_vals, W, b, node_idx, pair_i, pair_j):
    n, d = features.shape
    p = cos_vals.shape[0]
    zeros = jnp.zeros(((n // 2 // NS) // 8 * 8, d), jnp.float32)

    sc_fn = pl.kernel(
        functools.partial(_sc_body, n, p),
        out_type=jax.ShapeDtypeStruct((n, d), jnp.float32),
        mesh=plsc.VectorSubcoreMesh(
            core_axis_name="c", subcore_axis_name="s",
            num_cores=NC, num_subcores=NS),
        compiler_params=pltpu.CompilerParams(needs_layout_passes=False),
        scratch_types=[
            pltpu.VMEM((2, C), jnp.int32),
            pltpu.VMEM((2, C), jnp.int32),
            pltpu.VMEM((2, C), jnp.int32),
            pltpu.VMEM((2, C), jnp.float32),
            pltpu.VMEM((2, C), jnp.int32),
            pltpu.VMEM((2, C + LANES), jnp.int32),
            pltpu.VMEM((C,), jnp.float32),
            pltpu.VMEM((C,), jnp.float32),
            pltpu.VMEM((C, d), jnp.float32),
            pltpu.VMEM((C, d), jnp.float32),
            pltpu.VMEM((2, C, d), jnp.float32),
            pltpu.VMEM((RING, n), jnp.float32),
            pltpu.VMEM((LANES,), jnp.int32),
            pltpu.VMEM_SHARED((n // 2, d), jnp.float32),
            pltpu.SemaphoreType.DMA,
            pltpu.SemaphoreType.DMA,
            pltpu.SemaphoreType.DMA,
            pltpu.SemaphoreType.DMA,
            pltpu.SemaphoreType.DMA,
        ],
    )
    agg = sc_fn(features, dist, cos_vals, node_idx, pair_i, pair_j, zeros)

    blk = 1000
    out = pl.pallas_call(
        _proj_body,
        grid=(n // blk,),
        in_specs=[
            pl.BlockSpec((blk, d), lambda i: (i, 0)),
            pl.BlockSpec((d, d), lambda i: (0, 0)),
            pl.BlockSpec((1, d), lambda i: (0, 0)),
        ],
        out_specs=pl.BlockSpec((blk, d), lambda i: (i, 0)),
        out_shape=jax.ShapeDtypeStruct((n, d), jnp.float32),
    )(agg, W, b.reshape(1, d))
    return out
